# baseline (speedup 1.0000x reference)
import functools

import numpy as np
import jax
import jax.numpy as jnp
from jax import lax
from jax.experimental import pallas as pl
from jax.experimental.pallas import tpu as pltpu
from jax.experimental.pallas import tpu_sc as plsc

_L = 16


def _shuffle(v, s):
    perm = lax.iota(jnp.int32, _L) ^ s
    dnums = lax.GatherDimensionNumbers(
        offset_dims=(), collapsed_slice_dims=(0,), start_index_map=(0,))
    return lax.gather(v, perm[:, None], dnums, slice_sizes=(1,),
                      mode=lax.GatherScatterMode.PROMISE_IN_BOUNDS)


def _xl_min(v):
    for s in (1, 2, 4, 8):
        v = jnp.minimum(v, _shuffle(v, s))
    return v


def _xl_max(v):
    for s in (1, 2, 4, 8):
        v = jnp.maximum(v, _shuffle(v, s))
    return v


def _cdist_body(x_ref, d_ref):
    x = x_ref[:, 0, :]
    n = x.shape[0]
    g = lax.dot_general(x, x, (((1,), (1,)), ((), ())),
                        preferred_element_type=jnp.float32,
                        precision=lax.Precision.HIGHEST)
    rc = (lax.broadcasted_iota(jnp.int32, (n, n), 0)
          - lax.broadcasted_iota(jnp.int32, (n, n), 1))
    eyef = jnp.maximum(1.0 - jnp.abs(rc).astype(jnp.float32), 0.0)
    ge = g * eyef
    ncol = jnp.sum(ge, axis=1, keepdims=True)
    nrow = jnp.sum(ge, axis=0, keepdims=True)
    d2 = jnp.maximum(ncol + nrow - 2.0 * g, 0.0)
    d_ref[...] = jnp.sqrt(d2)


def _cdist(x):
    n = x.shape[0]
    return pl.pallas_call(
        _cdist_body,
        out_shape=jax.ShapeDtypeStruct((n, n), jnp.float32),
    )(x)


def _sc_retrieve(d_mat, labels):
    B = d_mat.shape[0]
    NC = B // _L
    mesh = plsc.VectorSubcoreMesh(core_axis_name="c", subcore_axis_name="s",
                                  num_cores=1, num_subcores=1)

    @functools.partial(
        pl.kernel,
        out_type=jax.ShapeDtypeStruct((B,), jnp.float32),
        mesh=mesh,
        scratch_types=[
            pltpu.VMEM((B, B), jnp.float32),
            pltpu.VMEM((B,), jnp.int32),
            pltpu.VMEM((B,), jnp.int32),
            pltpu.VMEM((B,), jnp.float32),
        ],
    )
    def sc_kernel(d_hbm, lab_hbm, out_hbm, d_v, lab_v, pk_v, pred_v):
        cid = lax.axis_index("c")
        sid = lax.axis_index("s")

        @pl.when((cid == 0) & (sid == 0))
        def _():
            pltpu.sync_copy(d_hbm, d_v)
            pltpu.sync_copy(lab_hbm, lab_v)
            iota = lax.iota(jnp.int32, _L)
            for c in range(NC):
                pk_v[pl.ds(c * _L, _L)] = (
                    ((iota + c * _L) << 8) | lab_v[pl.ds(c * _L, _L)])

            big = jnp.int32(2 ** 30)
            inf_v = jnp.full((_L,), jnp.inf, jnp.float32)

            p_regs = [inf_v] * NC
            p_regs[0] = jnp.where(iota == 0, jnp.float32(0.0), inf_v)
            min_d = inf_v
            max_d = jnp.zeros((_L,), jnp.float32)
            R = jnp.full((_L,), 1.0, jnp.float32)

            for b in range(NC):
                labc_f = lab_v[pl.ds(b * _L, _L)].astype(jnp.float32)
                predreg = jnp.zeros((_L,), jnp.float32)
                if b == 0:
                    predreg = jnp.where(iota == 0, labc_f, predreg)

                def step(i, carry, b=b, labc_f=labc_f, frozen=tuple(p_regs)):
                    min_d, max_d, R, predreg, pb = carry
                    pregs = list(frozen[:b]) + [pb]
                    mins = []
                    maxs = []
                    for c in range(b + 1):
                        row = d_v[i, pl.ds(c * _L, _L)]
                        pkc = pk_v[pl.ds(c * _L, _L)]
                        mins.append((row + pregs[c], pkc))
                        maxs.append(row - pregs[c])
                    while len(mins) > 1:
                        nm = []
                        for k in range(0, len(mins) - 1, 2):
                            va, pa = mins[k]
                            vb, pb = mins[k + 1]
                            nm.append((jnp.minimum(va, vb),
                                       jnp.where(vb < va, pb, pa)))
                        if len(mins) % 2:
                            nm.append(mins[-1])
                        mins = nm
                    while len(maxs) > 1:
                        nx = [jnp.maximum(maxs[k], maxs[k + 1])
                              for k in range(0, len(maxs) - 1, 2)]
                        if len(maxs) % 2:
                            nx.append(maxs[-1])
                        maxs = nx
                    runmin, runpk = mins[0]
                    runmax = maxs[0]
                    tmp_min = _xl_min(runmin)
                    tmp_max = _xl_max(runmax)
                    bestpk = _xl_min(jnp.where(runmin == tmp_min, runpk, big))
                    lab_nn = (bestpk & 255).astype(jnp.float32)

                    append = tmp_min > R
                    min_d = jnp.minimum(tmp_min, min_d)
                    max_d = jnp.maximum(tmp_max, max_d)
                    R = (min_d + max_d) / 3.0

                    lanemask = iota == (i - b * _L)
                    predreg = jnp.where(
                        lanemask, jnp.where(append, labc_f, lab_nn), predreg)
                    pb = jnp.where(
                        lanemask, jnp.where(append, jnp.float32(0.0), pb),
                        pb)
                    return (min_d, max_d, R, predreg, pb)

                def pair(k, carry, b=b, step=step):
                    i0 = b * _L + 2 * k
                    return step(i0 + 1, step(i0, carry))

                carry = (min_d, max_d, R, predreg, p_regs[b])
                if b == 0:
                    carry = step(1, carry)
                    out = lax.fori_loop(1, _L // 2, pair, carry)
                else:
                    out = lax.fori_loop(0, _L // 2, pair, carry)
                min_d, max_d, R, predreg, p_regs[b] = out
                pred_v[pl.ds(b * _L, _L)] = predreg

            pltpu.sync_copy(pred_v, out_hbm)

    return sc_kernel(d_mat, labels)


def kernel(x, set_labels, update_ref_insts=True):
    B = x.shape[0]
    dist = _cdist(x.astype(jnp.float32))
    preds = _sc_retrieve(dist, set_labels.astype(jnp.int32))
    upd = jnp.asarray(update_ref_insts, dtype=bool)
    return jnp.where(upd, preds, jnp.float32(jnp.nan)).astype(jnp.float32)


# speedup vs baseline: 1.0107x; 1.0107x over previous
import functools

import numpy as np
import jax
import jax.numpy as jnp
from jax import lax
from jax.experimental import pallas as pl
from jax.experimental.pallas import tpu as pltpu
from jax.experimental.pallas import tpu_sc as plsc

_L = 16


def _shuffle(v, s):
    perm = lax.iota(jnp.int32, _L) ^ s
    dnums = lax.GatherDimensionNumbers(
        offset_dims=(), collapsed_slice_dims=(0,), start_index_map=(0,))
    return lax.gather(v, perm[:, None], dnums, slice_sizes=(1,),
                      mode=lax.GatherScatterMode.PROMISE_IN_BOUNDS)


def _xl_min(v):
    for s in (1, 2, 4, 8):
        v = jnp.minimum(v, _shuffle(v, s))
    return v


def _xl_max(v):
    for s in (1, 2, 4, 8):
        v = jnp.maximum(v, _shuffle(v, s))
    return v


def _cdist_body(x_ref, d_ref):
    x = x_ref[:, 0, :]
    n = x.shape[0]
    g = lax.dot_general(x, x, (((1,), (1,)), ((), ())),
                        preferred_element_type=jnp.float32,
                        precision=lax.Precision.HIGHEST)
    rc = (lax.broadcasted_iota(jnp.int32, (n, n), 0)
          - lax.broadcasted_iota(jnp.int32, (n, n), 1))
    eyef = jnp.maximum(1.0 - jnp.abs(rc).astype(jnp.float32), 0.0)
    ge = g * eyef
    ncol = jnp.sum(ge, axis=1, keepdims=True)
    nrow = jnp.sum(ge, axis=0, keepdims=True)
    d2 = jnp.maximum(ncol + nrow - 2.0 * g, 0.0)
    d_ref[...] = jnp.sqrt(d2)


def _cdist(x):
    n = x.shape[0]
    return pl.pallas_call(
        _cdist_body,
        out_shape=jax.ShapeDtypeStruct((n, n), jnp.float32),
    )(x)


def _sc_retrieve(d_mat, labels):
    B = d_mat.shape[0]
    NC = B // _L
    mesh = plsc.VectorSubcoreMesh(core_axis_name="c", subcore_axis_name="s",
                                  num_cores=1, num_subcores=1)

    @functools.partial(
        pl.kernel,
        out_type=jax.ShapeDtypeStruct((B,), jnp.float32),
        mesh=mesh,
        scratch_types=[
            pltpu.VMEM((B, B), jnp.float32),
            pltpu.VMEM((B,), jnp.int32),
            pltpu.VMEM((B,), jnp.int32),
            pltpu.VMEM((B,), jnp.float32),
        ],
    )
    def sc_kernel(d_hbm, lab_hbm, out_hbm, d_v, lab_v, pk_v, pred_v):
        cid = lax.axis_index("c")
        sid = lax.axis_index("s")

        @pl.when((cid == 0) & (sid == 0))
        def _():
            pltpu.sync_copy(d_hbm, d_v)
            pltpu.sync_copy(lab_hbm, lab_v)
            iota = lax.iota(jnp.int32, _L)
            for c in range(NC):
                pk_v[pl.ds(c * _L, _L)] = (
                    ((iota + c * _L) << 8) | lab_v[pl.ds(c * _L, _L)])

            big = jnp.int32(2 ** 30)
            inf_v = jnp.full((_L,), jnp.inf, jnp.float32)

            p_regs = [inf_v] * NC
            p_regs[0] = jnp.where(iota == 0, jnp.float32(0.0), inf_v)
            min_d = inf_v
            max_d = jnp.zeros((_L,), jnp.float32)
            R = jnp.full((_L,), 1.0, jnp.float32)

            for b in range(NC):
                labc_f = lab_v[pl.ds(b * _L, _L)].astype(jnp.float32)
                predreg = jnp.zeros((_L,), jnp.float32)
                if b == 0:
                    predreg = jnp.where(iota == 0, labc_f, predreg)

                def step(i, carry, b=b, labc_f=labc_f, frozen=tuple(p_regs)):
                    min_d, max_d, R, predreg, pb = carry
                    pregs = list(frozen[:b]) + [pb]
                    mins = []
                    maxs = []
                    for c in range(b + 1):
                        row = d_v[i, pl.ds(c * _L, _L)]
                        pkc = pk_v[pl.ds(c * _L, _L)]
                        mins.append((row + pregs[c], pkc))
                        maxs.append(row - pregs[c])
                    while len(mins) > 1:
                        nm = []
                        for k in range(0, len(mins) - 1, 2):
                            va, pa = mins[k]
                            vb, pb = mins[k + 1]
                            nm.append((jnp.minimum(va, vb),
                                       jnp.where(vb < va, pb, pa)))
                        if len(mins) % 2:
                            nm.append(mins[-1])
                        mins = nm
                    while len(maxs) > 1:
                        nx = [jnp.maximum(maxs[k], maxs[k + 1])
                              for k in range(0, len(maxs) - 1, 2)]
                        if len(maxs) % 2:
                            nx.append(maxs[-1])
                        maxs = nx
                    runmin, runpk = mins[0]
                    runmax = maxs[0]
                    tmp_min = _xl_min(runmin)
                    tmp_max = _xl_max(runmax)
                    bestpk = _xl_min(jnp.where(runmin == tmp_min, runpk, big))
                    lab_nn = (bestpk & 255).astype(jnp.float32)

                    append = tmp_min > R
                    min_d = jnp.minimum(tmp_min, min_d)
                    max_d = jnp.maximum(tmp_max, max_d)
                    R = (min_d + max_d) / 3.0

                    lanemask = iota == (i - b * _L)
                    predreg = jnp.where(
                        lanemask, jnp.where(append, labc_f, lab_nn), predreg)
                    pb = jnp.where(
                        lanemask, jnp.where(append, jnp.float32(0.0), pb),
                        pb)
                    return (min_d, max_d, R, predreg, pb)

                lo = 1 if b == 0 else b * _L
                out = lax.fori_loop(lo, b * _L + _L, step,
                                    (min_d, max_d, R, predreg, p_regs[b]))
                min_d, max_d, R, predreg, p_regs[b] = out
                pred_v[pl.ds(b * _L, _L)] = predreg

            pltpu.sync_copy(pred_v, out_hbm)

    return sc_kernel(d_mat, labels)


def kernel(x, set_labels, update_ref_insts=True):
    B = x.shape[0]
    dist = _cdist(x.astype(jnp.float32))
    preds = _sc_retrieve(dist, set_labels.astype(jnp.int32))
    upd = jnp.asarray(update_ref_insts, dtype=bool)
    return jnp.where(upd, preds, jnp.float32(jnp.nan)).astype(jnp.float32)
